# baseline (speedup 1.0000x reference)
import functools

import jax
import jax.numpy as jnp
from jax import lax
from jax.experimental import pallas as pl
from jax.experimental.pallas import tpu as pltpu
from jax.experimental.pallas import tpu_sc as plsc

B, C, H, W = 4, 3, 224, 224
N_SEG = 196
EMBED = 768
N_PIX = H * W
NC, NS, L = 2, 16, 16
NW = NC * NS
W_PER_B = NW // B
SEG_PAD = 224
COMP = 8
ACC_COLS = L * COMP
UNROLL = 2
ROWS = H // W_PER_B
VPR = W // L


def _sc_partial_sums(img, segments, zeros2d):
    mesh = plsc.VectorSubcoreMesh(core_axis_name="c", subcore_axis_name="s")

    @functools.partial(
        pl.kernel,
        out_type=jax.ShapeDtypeStruct((NW, SEG_PAD, ACC_COLS), jnp.float32),
        mesh=mesh,
        compiler_params=pltpu.CompilerParams(needs_layout_passes=False),
        scratch_types=[
            pltpu.VMEM((ROWS + 4, W), jnp.float32),
            pltpu.VMEM((ROWS + 4, W), jnp.float32),
            pltpu.VMEM((ROWS + 4, W), jnp.float32),
            pltpu.VMEM((ROWS + 4, W), jnp.int32),
            pltpu.VMEM((SEG_PAD, ACC_COLS), jnp.float32),
            pltpu.VMEM((SEG_PAD, ACC_COLS), jnp.float32),
            pltpu.SemaphoreType.DMA,
        ],
    )
    def k(img_hbm, seg_hbm, z_hbm, out_hbm,
          c0_v, c1_v, c2_v, sg_v, acc, acc2, sem):
        wid = lax.axis_index("c") * NS + lax.axis_index("s")
        bi = wid // W_PER_B
        si = wid % W_PER_B
        r0 = si * ROWS
        off = (si % 2) * 4
        a0 = pl.multiple_of(r0 - off, 8)

        cps = [
            pltpu.async_copy(img_hbm.at[bi, 0, pl.ds(a0, ROWS + 4)], c0_v, sem),
            pltpu.async_copy(img_hbm.at[bi, 1, pl.ds(a0, ROWS + 4)], c1_v, sem),
            pltpu.async_copy(img_hbm.at[bi, 2, pl.ds(a0, ROWS + 4)], c2_v, sem),
            pltpu.async_copy(seg_hbm.at[bi, pl.ds(a0, ROWS + 4)], sg_v, sem),
            pltpu.async_copy(z_hbm, acc, sem),
            pltpu.async_copy(z_hbm, acc2, sem),
        ]
        for cp in cps:
            cp.wait()

        lane = lax.broadcasted_iota(jnp.int32, (L,), 0)
        cols = lane * COMP
        ones = jnp.ones((L,), jnp.float32)
        lane_f = lane.astype(jnp.float32)
        inv = jnp.float32(1.0 / (W - 1))
        xvecs = [(lane_f + (j * L)) * inv for j in range(VPR)]

        @plsc.parallel_loop(0, ROWS, 1, unroll=UNROLL)
        def _(r):
            ro = off + r
            yval = (r0 + r).astype(jnp.float32) * inv
            yvec = jnp.full((L,), 1.0, jnp.float32) * yval
            for j in range(VPR):
                a = acc if j % 2 == 0 else acc2
                sl = pl.ds(j * L, L)
                seg = sg_v[ro, sl]
                plsc.addupdate_scatter(a, [seg, cols], c0_v[ro, sl])
                plsc.addupdate_scatter(a, [seg, cols + 1], c1_v[ro, sl])
                plsc.addupdate_scatter(a, [seg, cols + 2], c2_v[ro, sl])
                plsc.addupdate_scatter(a, [seg, cols + 3], xvecs[j])
                plsc.addupdate_scatter(a, [seg, cols + 4], yvec)
                plsc.addupdate_scatter(a, [seg, cols + 5], ones)

        @plsc.parallel_loop(0, SEG_PAD, 1, unroll=2)
        def _(s):
            for kk in range(ACC_COLS // L):
                csl = pl.ds(kk * L, L)
                acc[s, csl] += acc2[s, csl]

        pltpu.sync_copy(acc, out_hbm.at[wid])

    return k(img, segments, zeros2d)


def _tc_finish_body(p_ref, wrep_ref, sel_ref, o_ref):
    total = jnp.sum(p_ref[...], axis=0)
    acc = total[:N_SEG, :]
    mm = lax.dot_general(acc, wrep_ref[...],
                         (((1,), (0,)), ((), ())),
                         preferred_element_type=jnp.float32)
    cnt = jnp.sum(acc * sel_ref[...], axis=1, keepdims=True)
    o_ref[0] = mm / jnp.maximum(cnt, 1.0)


def _tc_finish(partials, W_lin, b_lin):
    wrep = jnp.zeros((L, COMP, EMBED), jnp.float32)
    wrep = wrep.at[:, :5, :].set(W_lin[None, :, :])
    wrep = wrep.at[:, 5, :].set(b_lin[None, :])
    wrep = wrep.reshape(ACC_COLS, EMBED)
    sel = jnp.zeros((L, COMP), jnp.float32).at[:, 5].set(1.0).reshape(1, ACC_COLS)
    return pl.pallas_call(
        _tc_finish_body,
        grid=(B,),
        in_specs=[
            pl.BlockSpec((W_PER_B, SEG_PAD, ACC_COLS), lambda i: (i, 0, 0)),
            pl.BlockSpec((ACC_COLS, EMBED), lambda i: (0, 0)),
            pl.BlockSpec((1, ACC_COLS), lambda i: (0, 0)),
        ],
        out_specs=pl.BlockSpec((1, N_SEG, EMBED), lambda i: (i, 0, 0)),
        out_shape=jax.ShapeDtypeStruct((B, N_SEG, EMBED), jnp.float32),
    )(partials, wrep, sel)


def kernel(img, segments, W_lin, b_lin):
    zeros2d = jnp.zeros((SEG_PAD, ACC_COLS), jnp.float32)
    partials = _sc_partial_sums(img, segments, zeros2d)
    return _tc_finish(partials, W_lin, b_lin)


# speedup vs baseline: 1.2264x; 1.2264x over previous
import functools

import jax
import jax.numpy as jnp
from jax import lax
from jax.experimental import pallas as pl
from jax.experimental.pallas import tpu as pltpu
from jax.experimental.pallas import tpu_sc as plsc

B, C, H, W = 4, 3, 224, 224
N_SEG = 196
EMBED = 768
N_PIX = H * W
NC, NS, L = 2, 16, 16
NW = NC * NS
W_PER_B = NW // B
SEG_PAD = 224
COMP = 8
ACC_COLS = L * COMP
UNROLL = 2
ROWS = H // W_PER_B
VPR = W // L


def _sc_partial_sums(img, segments):
    mesh = plsc.VectorSubcoreMesh(core_axis_name="c", subcore_axis_name="s")

    @functools.partial(
        pl.kernel,
        out_type=jax.ShapeDtypeStruct((NW, SEG_PAD, ACC_COLS), jnp.float32),
        mesh=mesh,
        compiler_params=pltpu.CompilerParams(needs_layout_passes=False),
        scratch_types=[
            pltpu.VMEM((ROWS + 4, W), jnp.float32),
            pltpu.VMEM((ROWS + 4, W), jnp.float32),
            pltpu.VMEM((ROWS + 4, W), jnp.float32),
            pltpu.VMEM((ROWS + 4, W), jnp.int32),
            pltpu.VMEM((SEG_PAD, ACC_COLS), jnp.float32),
            pltpu.SemaphoreType.DMA,
        ],
    )
    def k(img_hbm, seg_hbm, out_hbm,
          c0_v, c1_v, c2_v, sg_v, acc, sem):
        wid = lax.axis_index("c") * NS + lax.axis_index("s")
        bi = wid // W_PER_B
        si = wid % W_PER_B
        r0 = si * ROWS
        off = (si % 2) * 4
        a0 = pl.multiple_of(r0 - off, 8)

        cps = [
            pltpu.async_copy(img_hbm.at[bi, 0, pl.ds(a0, ROWS + 4)], c0_v, sem),
            pltpu.async_copy(img_hbm.at[bi, 1, pl.ds(a0, ROWS + 4)], c1_v, sem),
            pltpu.async_copy(img_hbm.at[bi, 2, pl.ds(a0, ROWS + 4)], c2_v, sem),
            pltpu.async_copy(seg_hbm.at[bi, pl.ds(a0, ROWS + 4)], sg_v, sem),
        ]

        zeros = jnp.zeros((L,), jnp.float32)

        @plsc.parallel_loop(0, SEG_PAD, 1, unroll=4)
        def _(s):
            for kk in range(ACC_COLS // L):
                acc[s, pl.ds(kk * L, L)] = zeros

        for cp in cps:
            cp.wait()

        lane = lax.broadcasted_iota(jnp.int32, (L,), 0)
        cols = lane * COMP
        ones = jnp.ones((L,), jnp.float32)
        lane_f = lane.astype(jnp.float32)
        inv = jnp.float32(1.0 / (W - 1))
        xvecs = [(lane_f + (j * L)) * inv for j in range(VPR)]

        @plsc.parallel_loop(0, ROWS, 1, unroll=UNROLL)
        def _(r):
            ro = off + r
            yval = (r0 + r).astype(jnp.float32) * inv
            yvec = jnp.full((L,), 1.0, jnp.float32) * yval
            for j in range(VPR):
                sl = pl.ds(j * L, L)
                seg = sg_v[ro, sl]
                plsc.addupdate_scatter(acc, [seg, cols], c0_v[ro, sl])
                plsc.addupdate_scatter(acc, [seg, cols + 1], c1_v[ro, sl])
                plsc.addupdate_scatter(acc, [seg, cols + 2], c2_v[ro, sl])
                plsc.addupdate_scatter(acc, [seg, cols + 3], xvecs[j])
                plsc.addupdate_scatter(acc, [seg, cols + 4], yvec)
                plsc.addupdate_scatter(acc, [seg, cols + 5], ones)

        pltpu.sync_copy(acc, out_hbm.at[wid])

    return k(img, segments)


def _tc_finish_body(p_ref, wrep_ref, sel_ref, o_ref):
    total = jnp.sum(p_ref[...], axis=0)
    acc = total[:N_SEG, :]
    mm = lax.dot_general(acc, wrep_ref[...],
                         (((1,), (0,)), ((), ())),
                         preferred_element_type=jnp.float32)
    cnt = jnp.sum(acc * sel_ref[...], axis=1, keepdims=True)
    o_ref[0] = mm / jnp.maximum(cnt, 1.0)


def _tc_finish(partials, W_lin, b_lin):
    wrep = jnp.zeros((L, COMP, EMBED), jnp.float32)
    wrep = wrep.at[:, :5, :].set(W_lin[None, :, :])
    wrep = wrep.at[:, 5, :].set(b_lin[None, :])
    wrep = wrep.reshape(ACC_COLS, EMBED)
    sel = jnp.zeros((L, COMP), jnp.float32).at[:, 5].set(1.0).reshape(1, ACC_COLS)
    return pl.pallas_call(
        _tc_finish_body,
        grid=(B,),
        in_specs=[
            pl.BlockSpec((W_PER_B, SEG_PAD, ACC_COLS), lambda i: (i, 0, 0)),
            pl.BlockSpec((ACC_COLS, EMBED), lambda i: (0, 0)),
            pl.BlockSpec((1, ACC_COLS), lambda i: (0, 0)),
        ],
        out_specs=pl.BlockSpec((1, N_SEG, EMBED), lambda i: (i, 0, 0)),
        out_shape=jax.ShapeDtypeStruct((B, N_SEG, EMBED), jnp.float32),
    )(partials, wrep, sel)


def kernel(img, segments, W_lin, b_lin):
    partials = _sc_partial_sums(img, segments)
    return _tc_finish(partials, W_lin, b_lin)


# speedup vs baseline: 1.2363x; 1.0081x over previous
import functools

import jax
import jax.numpy as jnp
from jax import lax
from jax.experimental import pallas as pl
from jax.experimental.pallas import tpu as pltpu
from jax.experimental.pallas import tpu_sc as plsc

B, C, H, W = 4, 3, 224, 224
N_SEG = 196
EMBED = 768
N_PIX = H * W
NC, NS, L = 2, 16, 16
NW = NC * NS
W_PER_B = NW // B
SEG_PAD = 200
COMP = 8
ACC_COLS = L * COMP
UNROLL = 2
ROWS = H // W_PER_B
VPR = W // L


def _sc_partial_sums(img, segments):
    mesh = plsc.VectorSubcoreMesh(core_axis_name="c", subcore_axis_name="s")

    @functools.partial(
        pl.kernel,
        out_type=jax.ShapeDtypeStruct((NW, SEG_PAD, ACC_COLS), jnp.float32),
        mesh=mesh,
        compiler_params=pltpu.CompilerParams(needs_layout_passes=False),
        scratch_types=[
            pltpu.VMEM((ROWS + 4, W), jnp.float32),
            pltpu.VMEM((ROWS + 4, W), jnp.float32),
            pltpu.VMEM((ROWS + 4, W), jnp.float32),
            pltpu.VMEM((ROWS + 4, W), jnp.int32),
            pltpu.VMEM((SEG_PAD, ACC_COLS), jnp.float32),
            pltpu.SemaphoreType.DMA,
        ],
    )
    def k(img_hbm, seg_hbm, out_hbm,
          c0_v, c1_v, c2_v, sg_v, acc, sem):
        wid = lax.axis_index("c") * NS + lax.axis_index("s")
        bi = wid // W_PER_B
        si = wid % W_PER_B
        r0 = si * ROWS
        off = (si % 2) * 4
        a0 = pl.multiple_of(r0 - off, 8)

        cps = [
            pltpu.async_copy(img_hbm.at[bi, 0, pl.ds(a0, ROWS + 4)], c0_v, sem),
            pltpu.async_copy(img_hbm.at[bi, 1, pl.ds(a0, ROWS + 4)], c1_v, sem),
            pltpu.async_copy(img_hbm.at[bi, 2, pl.ds(a0, ROWS + 4)], c2_v, sem),
            pltpu.async_copy(seg_hbm.at[bi, pl.ds(a0, ROWS + 4)], sg_v, sem),
        ]

        zeros = jnp.zeros((L,), jnp.float32)

        @plsc.parallel_loop(0, SEG_PAD, 1, unroll=4)
        def _(s):
            for kk in range(ACC_COLS // L):
                acc[s, pl.ds(kk * L, L)] = zeros

        for cp in cps:
            cp.wait()

        lane = lax.broadcasted_iota(jnp.int32, (L,), 0)
        cols = lane * COMP
        ones = jnp.ones((L,), jnp.float32)
        lane_f = lane.astype(jnp.float32)
        inv = jnp.float32(1.0 / (W - 1))
        xvecs = [(lane_f + (j * L)) * inv for j in range(VPR)]

        @plsc.parallel_loop(0, ROWS, 1, unroll=UNROLL)
        def _(r):
            ro = off + r
            yval = (r0 + r).astype(jnp.float32) * inv
            yvec = jnp.full((L,), 1.0, jnp.float32) * yval
            for j in range(VPR):
                sl = pl.ds(j * L, L)
                seg = sg_v[ro, sl]
                plsc.addupdate_scatter(acc, [seg, cols], c0_v[ro, sl])
                plsc.addupdate_scatter(acc, [seg, cols + 1], c1_v[ro, sl])
                plsc.addupdate_scatter(acc, [seg, cols + 2], c2_v[ro, sl])
                plsc.addupdate_scatter(acc, [seg, cols + 3], xvecs[j])
                plsc.addupdate_scatter(acc, [seg, cols + 4], yvec)
                plsc.addupdate_scatter(acc, [seg, cols + 5], ones)

        pltpu.sync_copy(acc, out_hbm.at[wid])

    return k(img, segments)


def _tc_finish_body(p_ref, wrep_ref, sel_ref, o_ref):
    total = jnp.sum(p_ref[...], axis=0)
    acc = total[:N_SEG, :]
    mm = lax.dot_general(acc, wrep_ref[...],
                         (((1,), (0,)), ((), ())),
                         preferred_element_type=jnp.float32)
    cnt = jnp.sum(acc * sel_ref[...], axis=1, keepdims=True)
    o_ref[0] = mm / jnp.maximum(cnt, 1.0)


def _tc_finish(partials, W_lin, b_lin):
    wrep = jnp.zeros((L, COMP, EMBED), jnp.float32)
    wrep = wrep.at[:, :5, :].set(W_lin[None, :, :])
    wrep = wrep.at[:, 5, :].set(b_lin[None, :])
    wrep = wrep.reshape(ACC_COLS, EMBED)
    sel = jnp.zeros((L, COMP), jnp.float32).at[:, 5].set(1.0).reshape(1, ACC_COLS)
    return pl.pallas_call(
        _tc_finish_body,
        grid=(B,),
        in_specs=[
            pl.BlockSpec((W_PER_B, SEG_PAD, ACC_COLS), lambda i: (i, 0, 0)),
            pl.BlockSpec((ACC_COLS, EMBED), lambda i: (0, 0)),
            pl.BlockSpec((1, ACC_COLS), lambda i: (0, 0)),
        ],
        out_specs=pl.BlockSpec((1, N_SEG, EMBED), lambda i: (i, 0, 0)),
        out_shape=jax.ShapeDtypeStruct((B, N_SEG, EMBED), jnp.float32),
    )(partials, wrep, sel)


def kernel(img, segments, W_lin, b_lin):
    partials = _sc_partial_sums(img, segments)
    return _tc_finish(partials, W_lin, b_lin)
